# baseline (speedup 1.0000x reference)
import functools

import jax
import jax.numpy as jnp
from jax import lax
from jax.experimental import pallas as pl
from jax.experimental.pallas import tpu as pltpu
from jax.experimental.pallas import tpu_sc as plsc

VOCAB = 1000000
EMBED_DIM = 64
BATCH = 1024
SEQ_LEN = 200

_INFO = plsc.get_sparse_core_info()
_NC, _NS, _L = _INFO.num_cores, _INFO.num_subcores, _INFO.num_lanes
_NW = _NC * _NS
_B = BATCH * SEQ_LEN
_G = 8
_BB = 128
_NBLK = BATCH // _BB
_NUNIT = SEQ_LEN * _NBLK
_UPW = _NUNIT // _NW
_NGRP = _BB // _L
_CG = EMBED_DIM // _L


def _sinusoid_table(seq_len, d):
    pos = jnp.arange(seq_len, dtype=jnp.float32)[:, None]
    i = jnp.arange(d, dtype=jnp.float32)[None, :]
    angle = pos / jnp.power(10000.0, 2.0 * jnp.floor(i / 2.0) / d)
    even_mask = (jnp.arange(d) % 2 == 0)[None, :]
    return jnp.where(even_mask, jnp.sin(angle), jnp.cos(angle))


@functools.partial(
    pl.kernel,
    mesh=plsc.VectorSubcoreMesh(core_axis_name="c", subcore_axis_name="s"),
    out_type=jax.ShapeDtypeStruct(
        (SEQ_LEN, _G, _NBLK, _G, _BB), jnp.float32),
    name="embed_gather_sc",
    scratch_types=[
        [pltpu.VMEM((_BB,), jnp.int32) for _ in range(2)],
        pltpu.VMEM((SEQ_LEN * EMBED_DIM,), jnp.float32),
        [pltpu.VMEM((_BB // _G, _G, EMBED_DIM), jnp.float32)
         for _ in range(2)],
        [pltpu.VMEM((_G, _G, _BB), jnp.float32) for _ in range(2)],
        [pltpu.SemaphoreType.DMA for _ in range(2)],
        [pltpu.SemaphoreType.DMA for _ in range(2)],
        [pltpu.SemaphoreType.DMA for _ in range(2)],
    ],
    compiler_params=pltpu.CompilerParams(use_tc_tiling_on_sc=True,
                                         needs_layout_passes=False),
)
def _embed_sc(table_hbm, idx_hbm, pe_hbm, out_hbm,
              idx_vms, pe_v, rows, blks, isems, gsems, ssems):
    wid = lax.axis_index("s") * _NC + lax.axis_index("c")
    ubase = wid * _UPW
    pltpu.sync_copy(pe_hbm, pe_v)

    scale = jnp.float32(EMBED_DIM ** -0.5)
    iot = lax.iota(jnp.int32, _L)
    i1v = lax.bitwise_and(iot, 7)
    i0vs = [cg * 2 + lax.shift_right_logical(iot, 3)
            for cg in range(_CG)]

    def fire_idx(b, u):
        pltpu.async_copy(
            idx_hbm.at[pl.ds((ubase + u) * _BB, _BB)], idx_vms[b], isems[b])

    def wait_idx(b):
        pltpu.make_async_copy(idx_hbm.at[pl.ds(0, _BB)],
                              idx_vms[b], isems[b]).wait()

    def issue_gathers(b):
        idx_vm = idx_vms[b]
        row = rows[b]
        sem = gsems[b]

        def body(k, carry):
            v = idx_vm[pl.ds(k * _L, _L)]
            vg = lax.shift_right_logical(v, 3)
            vs = lax.bitwise_and(v, 7)
            for lane in range(_L):
                dg = 2 * k + lane // _G
                pltpu.async_copy(table_hbm.at[vg[lane], vs[lane], :],
                                 row.at[dg, lane % _G, :], sem)
            return carry
        lax.fori_loop(0, _NGRP, body, 0)

    def drain_gathers(b):
        pltpu.make_async_copy(table_hbm.at[pl.ds(0, _BB // _G), :, :],
                              rows[b], gsems[b]).wait()

    def compute(b, u):
        row = rows[b]
        blk = blks[b]
        p = lax.shift_right_logical(ubase + u, 3)
        pe_cgs = [pe_v[pl.ds(p * EMBED_DIM + cg * _L, _L)]
                  for cg in range(_CG)]

        def body(r, carry):
            dg = lax.shift_right_logical(r, 3)
            s = lax.bitwise_and(r, 7)
            bvec = iot * 0 + r
            for cg in range(_CG):
                x = row[dg, s, pl.ds(cg * _L, _L)]
                y = x * scale + pe_cgs[cg]
                plsc.store_scatter(blk, [i0vs[cg], i1v, bvec], y)
            return carry
        lax.fori_loop(0, _BB, body, 0)

    def fire_store(b, u):
        g = ubase + u
        p = lax.shift_right_logical(g, 3)
        bblk = lax.bitwise_and(g, 7)
        pltpu.async_copy(blks[b], out_hbm.at[p, :, bblk, :, :], ssems[b])

    def wait_store(b):
        pltpu.make_async_copy(blks[b], out_hbm.at[0, :, 0, :, :],
                              ssems[b]).wait()

    fire_idx(0, 0)
    fire_idx(1, 1)
    wait_idx(0)
    issue_gathers(0)
    fire_idx(0, 2)
    wait_idx(1)
    issue_gathers(1)
    fire_idx(1, 3)
    drain_gathers(0)
    compute(0, 0)
    fire_store(0, 0)

    def pair_body(q, carry):
        for b in range(2):
            u = 2 * q + b
            wait_store(b)
            wait_idx(b)
            issue_gathers(b)
            fire_idx(b, u + 2)
            drain_gathers(1 - b)
            compute(1 - b, u - 1)
            fire_store(1 - b, u - 1)
        return carry
    lax.fori_loop(1, _UPW // 2 - 1, pair_body, 0)

    for u in (_UPW - 2, _UPW - 1):
        b = u % 2
        wait_store(b)
        wait_idx(b)
        issue_gathers(b)
        drain_gathers(1 - b)
        compute(1 - b, u - 1)
        fire_store(1 - b, u - 1)
    drain_gathers((_UPW - 1) % 2)
    compute((_UPW - 1) % 2, _UPW - 1)
    fire_store((_UPW - 1) % 2, _UPW - 1)
    wait_store(0)
    wait_store(1)


def kernel(input, table):
    idx_t = input.T.reshape(-1).astype(jnp.int32)
    pe = _sinusoid_table(SEQ_LEN, EMBED_DIM)
    table3 = table.reshape(VOCAB // _G, _G, EMBED_DIM)
    out5 = _embed_sc(table3, idx_t, pe.reshape(-1))
    return out5.transpose(2, 4, 0, 1, 3).reshape(BATCH, SEQ_LEN, EMBED_DIM)


# speedup vs baseline: 1.2768x; 1.2768x over previous
import functools

import jax
import jax.numpy as jnp
from jax import lax
from jax.experimental import pallas as pl
from jax.experimental.pallas import tpu as pltpu
from jax.experimental.pallas import tpu_sc as plsc

VOCAB = 1000000
EMBED_DIM = 64
BATCH = 1024
SEQ_LEN = 200

_INFO = plsc.get_sparse_core_info()
_NC, _NS, _L = _INFO.num_cores, _INFO.num_subcores, _INFO.num_lanes
_NW = _NC * _NS
_B = BATCH * SEQ_LEN
_BPW = _B // _NW
_CHUNK = 2 * SEQ_LEN
_NCHUNK = _BPW // _CHUNK
_NGRP = _CHUNK // 16
_CG = EMBED_DIM // _L
_NBUF = 2
_G = 8
_CHG = _CHUNK // _G


def _sinusoid_table(seq_len, d):
    pos = jnp.arange(seq_len, dtype=jnp.float32)[:, None]
    i = jnp.arange(d, dtype=jnp.float32)[None, :]
    angle = pos / jnp.power(10000.0, 2.0 * jnp.floor(i / 2.0) / d)
    even_mask = (jnp.arange(d) % 2 == 0)[None, :]
    return jnp.where(even_mask, jnp.sin(angle), jnp.cos(angle))


@functools.partial(
    pl.kernel,
    mesh=plsc.VectorSubcoreMesh(core_axis_name="c", subcore_axis_name="s"),
    out_type=jax.ShapeDtypeStruct((_B // _G, _G, EMBED_DIM), jnp.float32),
    name="embed_gather_sc",
    scratch_types=[
        [pltpu.VMEM((_CHUNK,), jnp.int32) for _ in range(_NBUF)],
        pltpu.VMEM((SEQ_LEN * EMBED_DIM,), jnp.float32),
        [pltpu.VMEM((_CHG, _G, EMBED_DIM), jnp.float32) for _ in range(_NBUF)],
        [pltpu.SemaphoreType.DMA for _ in range(_NBUF)],
        [pltpu.SemaphoreType.DMA for _ in range(_NBUF)],
        [pltpu.SemaphoreType.DMA for _ in range(_NBUF)],
    ],
    compiler_params=pltpu.CompilerParams(use_tc_tiling_on_sc=True),
)
def _embed_sc(table_hbm, idx_hbm, pe_hbm, out_hbm,
              idx_vms, pe_v, bufs, isems, gsems, ssems):
    wid = lax.axis_index("s") * _NC + lax.axis_index("c")
    wbase = wid * _BPW
    pltpu.sync_copy(pe_hbm, pe_v)

    scale = jnp.float32(EMBED_DIM ** -0.5)
    half_g = SEQ_LEN // _G

    def fire_idx(b, c):
        pltpu.async_copy(
            idx_hbm.at[pl.ds(wbase + c * _CHUNK, _CHUNK)], idx_vms[b], isems[b])

    def wait_idx(b):
        pltpu.make_async_copy(idx_hbm.at[pl.ds(0, _CHUNK)],
                              idx_vms[b], isems[b]).wait()

    def issue_gathers(b):
        idx_vm = idx_vms[b]
        buf = bufs[b]
        sem = gsems[b]

        def body(k, carry):
            base = k * _L
            v = idx_vm[pl.ds(base, _L)]
            vg = lax.shift_right_logical(v, 3)
            vs = lax.bitwise_and(v, 7)
            for lane in range(_L):
                dg = 2 * k + lane // _G
                pltpu.async_copy(table_hbm.at[vg[lane], vs[lane], :],
                                 buf.at[dg, lane % _G, :], sem)
            return carry
        lax.fori_loop(0, _NGRP, body, 0)

    def drain_gathers(b):
        pltpu.make_async_copy(table_hbm.at[pl.ds(0, _CHG), :, :],
                              bufs[b], gsems[b]).wait()

    def compute(b):
        buf = bufs[b]

        def fma_body(q, carry):
            g = lax.shift_right_logical(q, 3)
            s = lax.bitwise_and(q, 7)
            for half in range(_CHUNK // SEQ_LEN):
                gg = g + half * half_g
                for cg in range(_CG):
                    sl = pl.ds(cg * _L, _L)
                    pe_row = pe_v[pl.ds(q * EMBED_DIM + cg * _L, _L)]
                    buf[gg, s, sl] = buf[gg, s, sl] * scale + pe_row
            return carry
        lax.fori_loop(0, SEQ_LEN, fma_body, 0)

    def fire_store(b, c):
        pltpu.async_copy(
            bufs[b],
            out_hbm.at[pl.ds((wbase + c * _CHUNK) // _G, _CHG), :, :],
            ssems[b])

    def wait_store(b):
        pltpu.make_async_copy(bufs[b],
                              out_hbm.at[pl.ds(0, _CHG), :, :], ssems[b]).wait()

    fire_idx(0, 0)
    fire_idx(1, 1)
    wait_idx(0)
    issue_gathers(0)
    fire_idx(0, 2)
    wait_idx(1)
    issue_gathers(1)
    fire_idx(1, 3)
    drain_gathers(0)
    compute(0)
    fire_store(0, 0)

    def pair_body(p, carry):
        for b in range(2):
            c = 2 * p + b
            wait_store(b)
            wait_idx(b)
            issue_gathers(b)
            fire_idx(b, c + 2)
            drain_gathers(1 - b)
            compute(1 - b)
            fire_store(1 - b, c - 1)
        return carry
    lax.fori_loop(1, _NCHUNK // 2 - 1, pair_body, 0)

    for c in (_NCHUNK - 2, _NCHUNK - 1):
        b = c % 2
        wait_store(b)
        wait_idx(b)
        issue_gathers(b)
        drain_gathers(1 - b)
        compute(1 - b)
        fire_store(1 - b, c - 1)
    drain_gathers((_NCHUNK - 1) % 2)
    compute((_NCHUNK - 1) % 2)
    fire_store((_NCHUNK - 1) % 2, _NCHUNK - 1)
    wait_store(0)
    wait_store(1)


def kernel(input, table):
    idx = input.reshape(-1).astype(jnp.int32)
    pe = _sinusoid_table(SEQ_LEN, EMBED_DIM)
    table3 = table.reshape(VOCAB // _G, _G, EMBED_DIM)
    out = _embed_sc(table3, idx, pe.reshape(-1))
    return out.reshape(BATCH, SEQ_LEN, EMBED_DIM)
